# baseline (speedup 1.0000x reference)
import dataclasses
import functools

import jax
import jax.numpy as jnp
from jax import lax
from jax.experimental import pallas as pl
from jax.experimental.pallas import tpu as pltpu
from jax.experimental.pallas import tpu_sc as plsc

N = 10000
E = 160000
D_IN = 256
H = 512
F = 128
NC = 2
NS = 16
NP = 10240
ROWS_PER_SUB = NP // NS
EP = 163840
CH = 128
NCH_D = EP // (NC * NS) // CH
NB = 5
BSH = 11
BR = 1 << BSH
AR = 2176
PCAP = 80
EPW = EP // (NC * NS)
VEC = 16

_mesh = plsc.VectorSubcoreMesh(
    core_axis_name="c", subcore_axis_name="s", num_cores=NC, num_subcores=NS
)

_cp = pltpu.CompilerParams()
if "needs_layout_passes" in pltpu.CompilerParams.__dataclass_fields__:
    _cp = dataclasses.replace(_cp, needs_layout_passes=False)


@functools.partial(
    pl.kernel,
    out_type=jax.ShapeDtypeStruct((NC, NP, F), jnp.float32),
    mesh=_mesh,
    scratch_types=[
        pltpu.VMEM((NCH_D, CH), jnp.int32),
        pltpu.VMEM((CH, F), jnp.float32),
        pltpu.VMEM_SHARED((NP, F), jnp.float32),
        pltpu.SemaphoreType.DMA,
    ],
)
def _sc_degree(dst_hbm, ones_hbm, zeros_hbm, out_hbm, dstm, onesv, acc, sem):
    c = lax.axis_index("c")
    s = lax.axis_index("s")
    pltpu.sync_copy(zeros_hbm, acc.at[pl.ds(s * ROWS_PER_SUB, ROWS_PER_SUB)])
    pltpu.sync_copy(ones_hbm, onesv)
    pltpu.sync_copy(dst_hbm.at[c * NS + s], dstm)
    plsc.subcore_barrier()
    @pl.loop(0, NCH_D)
    def _(i):
        pltpu.async_copy(onesv, acc.at[dstm.at[i]], sem, add=True)
    @pl.loop(0, NCH_D)
    def _(i):
        pltpu.make_async_copy(onesv, acc.at[dstm.at[i]], sem).wait()
    plsc.subcore_barrier()
    sl = pl.ds(s * ROWS_PER_SUB, ROWS_PER_SUB)
    pltpu.sync_copy(acc.at[sl], out_hbm.at[c].at[sl])


@functools.partial(
    pl.kernel,
    out_type=[
        jax.ShapeDtypeStruct((NC * NS, PCAP * CH), jnp.int32),
        jax.ShapeDtypeStruct((NC * NS, PCAP, CH), jnp.int32),
        jax.ShapeDtypeStruct((NC * NS, 2, VEC), jnp.int32),
    ],
    mesh=_mesh,
    scratch_types=[
        pltpu.VMEM((NCH_D, CH), jnp.int32),
        pltpu.VMEM((NCH_D, CH), jnp.int32),
        pltpu.VMEM((PCAP * CH,), jnp.int32),
        pltpu.VMEM((PCAP * CH,), jnp.int32),
        pltpu.VMEM((PCAP, CH), jnp.int32),
        pltpu.VMEM((2, VEC), jnp.int32),
        pltpu.VMEM((VEC,), jnp.int32),
        pltpu.VMEM((VEC,), jnp.int32),
    ],
    compiler_params=_cp,
)
def _sc_bin(src_hbm, dst_hbm, bsrc_hbm, brloc_hbm, meta_hbm,
            srcm, dstm, locs, locr, locr2, metav, cntv, curv):
    c = lax.axis_index("c")
    s = lax.axis_index("s")
    w = c * NS + s
    pltpu.sync_copy(src_hbm.at[w], srcm)
    pltpu.sync_copy(dst_hbm.at[w], dstm)
    lanes = lax.iota(jnp.int32, VEC)

    @pl.loop(0, PCAP * (CH // VEC))
    def _(i):
        locs.at[pl.ds(i * VEC, VEC)][...] = jnp.zeros((VEC,), jnp.int32)
        locr.at[pl.ds(i * VEC, VEC)][...] = BR + lanes

    cntv[...] = jnp.zeros((VEC,), jnp.int32)
    @pl.loop(0, EPW // VEC)
    def _(i):
        r = i // (CH // VEC)
        col = (i % (CH // VEC)) * VEC
        v = dstm.at[r, pl.ds(col, VEC)][...]
        b_of = lax.shift_right_logical(v, BSH)
        cnts = cntv[...]
        for b in range(NB):
            pop = plsc.all_reduce_population_count(b_of == b)
            cnts = jnp.where(lanes == b, cnts + pop, cnts)
        cntv[...] = cnts

    cnts = cntv[...]
    nch = lax.shift_right_logical(cnts + (CH - 1), 7)
    rnd8 = jnp.where(lanes < NB,
                     lax.shift_left(lax.shift_right_logical(nch + 7, 3), 3), 0)
    off8 = plsc.cumsum(rnd8) - rnd8
    metav.at[0, pl.ds(0, VEC)][...] = nch
    metav.at[1, pl.ds(0, VEC)][...] = off8
    curv[...] = jnp.where(lanes < NB, off8 * CH, 0)

    @pl.loop(0, EPW // VEC)
    def _(i):
        r = i // (CH // VEC)
        col = (i % (CH // VEC)) * VEC
        v = dstm.at[r, pl.ds(col, VEC)][...]
        sv = srcm.at[r, pl.ds(col, VEC)][...]
        b_of = lax.shift_right_logical(v, BSH)
        rloc = v - lax.shift_left(b_of, BSH)
        cur = curv[...]
        for b in range(NB):
            m = b_of == b
            base = jnp.max(jnp.where(lanes == b, cur, 0))
            cs = plsc.cumsum(m.astype(jnp.int32))
            idx = base + cs - 1
            plsc.store_scatter(locs, (idx,), sv, mask=m)
            plsc.store_scatter(locr, (idx,), rloc, mask=m)
            pop = plsc.all_reduce_population_count(m)
            cur = jnp.where(lanes == b, cur + pop, cur)
        curv[...] = cur

    @pl.loop(0, PCAP * (CH // VEC))
    def _(i):
        r = i // (CH // VEC)
        col = (i % (CH // VEC)) * VEC
        locr2.at[r, pl.ds(col, VEC)][...] = locr.at[pl.ds(i * VEC, VEC)][...]

    pltpu.sync_copy(locs, bsrc_hbm.at[w])
    pltpu.sync_copy(locr2, brloc_hbm.at[w])
    pltpu.sync_copy(metav, meta_hbm.at[w])


WIN = 8


@functools.partial(
    pl.kernel,
    out_type=jax.ShapeDtypeStruct((NC, NP, F), jnp.float32),
    mesh=_mesh,
    scratch_types=[
        pltpu.VMEM((WIN * CH,), jnp.int32),
        pltpu.VMEM((WIN, CH), jnp.int32),
        pltpu.VMEM((2, 2, VEC), jnp.int32),
        pltpu.VMEM((CH, F), jnp.float32),
        pltpu.VMEM_SHARED((NP, F), jnp.float32),
        pltpu.VMEM_SHARED((AR, F), jnp.float32),
    ],
    compiler_params=_cp,
)
def _sc_edge_pass(y_hbm, bsrc_hbm, brloc_hbm, meta_hbm, zeros_hbm, out_hbm,
                  srcw, rlocw, metv, rows0, ysp, acc):
    c = lax.axis_index("c")
    s = lax.axis_index("s")
    lanes = lax.iota(jnp.int32, VEC)

    sl = pl.ds(s * ROWS_PER_SUB, ROWS_PER_SUB)
    pltpu.sync_copy(y_hbm.at[c].at[sl], ysp.at[sl])
    pltpu.sync_copy(meta_hbm.at[pl.ds(2 * s, 2)], metv)

    for b in range(NB):
        pltpu.sync_copy(zeros_hbm, acc.at[pl.ds(s * (AR // NS), AR // NS)])
        plsc.subcore_barrier()
        for pi in range(2):
            p = 2 * s + pi
            nchv = metv.at[pi, 0, pl.ds(0, VEC)][...]
            offv = metv.at[pi, 1, pl.ds(0, VEC)][...]
            nch = jnp.max(jnp.where(lanes == b, nchv, 0))
            offc = pl.multiple_of(jnp.max(jnp.where(lanes == b, offv, 0)), 8)
            for win in range(5):
                @pl.when(nch > win * WIN)
                def _():
                    pltpu.sync_copy(
                        bsrc_hbm.at[p].at[pl.ds((offc + win * WIN) * CH,
                                                WIN * CH)], srcw)
                    pltpu.sync_copy(
                        brloc_hbm.at[p].at[pl.ds(offc + win * WIN, WIN)],
                        rlocw)
                    cnt = jnp.minimum(nch - win * WIN, WIN)
                    @pl.loop(0, WIN)
                    def _(k):
                        @pl.when(k < cnt)
                        def _():
                            pltpu.sync_copy(
                                ysp.at[srcw.at[pl.ds(k * CH, CH)]], rows0)
                            pltpu.sync_copy(rows0, acc.at[rlocw.at[k]],
                                            add=True)
        plsc.subcore_barrier()
        rps = BR // NS
        pltpu.sync_copy(acc.at[pl.ds(s * rps, rps)],
                        out_hbm.at[c].at[pl.ds(b * BR + s * rps, rps)])
        plsc.subcore_barrier()


BN = BR


def _dinv_block(cnt_ref):
    deg = cnt_ref[0, :, 0] + cnt_ref[1, :, 0] + 2.0
    return lax.rsqrt(deg)[:, None]


def _tc_scale_body(cnt_ref, x_ref, y_ref):
    dinv = _dinv_block(cnt_ref)
    y_ref[0] = dinv * x_ref[:, :F]
    y_ref[1] = dinv * x_ref[:, F:]


def _tc_mid_body(cnt_ref, s1_ref, x_ref, w1_ref, b1_ref, w2_ref,
                 y2_ref, h2_ref):
    dinv = _dinv_block(cnt_ref)
    sfull = jnp.concatenate([s1_ref[0], s1_ref[1]], axis=1)
    xa = dinv * sfull + (2.0 * dinv * dinv) * x_ref[...]
    t = jnp.dot(xa, w1_ref[...], preferred_element_type=jnp.float32)
    t = t + b1_ref[...]
    t = jnp.where(t > 0.0, t, jnp.exp(jnp.minimum(t, 0.0)) - 1.0)
    h2 = jnp.dot(t, w2_ref[...], preferred_element_type=jnp.float32)
    h2_ref[...] = h2
    y2 = dinv * h2
    y2_ref[0] = y2[:, :F]
    y2_ref[1] = y2[:, F:]


def _tc_final_body(cnt_ref, s2_ref, h2_ref, b2_ref, out_ref):
    dinv = _dinv_block(cnt_ref)
    sfull = jnp.concatenate([s2_ref[0], s2_ref[1]], axis=1)
    out_ref[...] = dinv * sfull + (2.0 * dinv * dinv) * h2_ref[...] + b2_ref[...]


def _cnt_spec():
    return pl.BlockSpec((NC, BN, F), lambda i: (0, i, 0))


def _s_spec():
    return pl.BlockSpec((NC, BN, F), lambda i: (0, i, 0))


@jax.jit
def kernel(x, edge_index, W1, b1, W2, b2):
    src = edge_index[0].astype(jnp.int32)
    dst = edge_index[1].astype(jnp.int32)
    pad = EP - E
    srcp = jnp.concatenate([src, jnp.zeros((pad,), jnp.int32)])
    junk = N + jnp.arange(pad, dtype=jnp.int32) % (NP - N)
    dstp = jnp.concatenate([dst, junk])
    src_d = srcp.reshape(NC * NS, NCH_D, CH)
    dst_d = dstp.reshape(NC * NS, NCH_D, CH)

    xp = jnp.zeros((NP, D_IN), x.dtype).at[:N].set(x)

    onesF = jnp.ones((CH, F), jnp.float32)
    zerosF = jnp.zeros((ROWS_PER_SUB, F), jnp.float32)
    zerosA = jnp.zeros((AR // NS, F), jnp.float32)

    cnt = _sc_degree(dst_d, onesF, zerosF)
    bsrc, brloc, meta = _sc_bin(src_d, dst_d)

    y = pl.pallas_call(
        _tc_scale_body,
        grid=(NB,),
        in_specs=[_cnt_spec(), pl.BlockSpec((BN, D_IN), lambda i: (i, 0))],
        out_specs=pl.BlockSpec((NC, BN, F), lambda i: (0, i, 0)),
        out_shape=jax.ShapeDtypeStruct((NC, NP, F), jnp.float32),
    )(cnt, xp)

    s1 = _sc_edge_pass(y, bsrc, brloc, meta, zerosA)

    y2, h2 = pl.pallas_call(
        _tc_mid_body,
        grid=(NB,),
        in_specs=[
            _cnt_spec(),
            _s_spec(),
            pl.BlockSpec((BN, D_IN), lambda i: (i, 0)),
            pl.BlockSpec((D_IN, H), lambda i: (0, 0)),
            pl.BlockSpec((1, H), lambda i: (0, 0)),
            pl.BlockSpec((H, D_IN), lambda i: (0, 0)),
        ],
        out_specs=[
            pl.BlockSpec((NC, BN, F), lambda i: (0, i, 0)),
            pl.BlockSpec((BN, D_IN), lambda i: (i, 0)),
        ],
        out_shape=[
            jax.ShapeDtypeStruct((NC, NP, F), jnp.float32),
            jax.ShapeDtypeStruct((NP, D_IN), jnp.float32),
        ],
    )(cnt, s1, xp, W1, b1.reshape(1, H), W2)

    s2 = _sc_edge_pass(y2, bsrc, brloc, meta, zerosA)

    out = pl.pallas_call(
        _tc_final_body,
        grid=(NB,),
        in_specs=[
            _cnt_spec(),
            _s_spec(),
            pl.BlockSpec((BN, D_IN), lambda i: (i, 0)),
            pl.BlockSpec((1, D_IN), lambda i: (0, 0)),
        ],
        out_specs=pl.BlockSpec((BN, D_IN), lambda i: (i, 0)),
        out_shape=jax.ShapeDtypeStruct((NP, D_IN), jnp.float32),
    )(cnt, s2, h2, b2.reshape(1, D_IN))
    return out[:N]


# speedup vs baseline: 1.1031x; 1.1031x over previous
import dataclasses
import functools

import jax
import jax.numpy as jnp
from jax import lax
from jax.experimental import pallas as pl
from jax.experimental.pallas import tpu as pltpu
from jax.experimental.pallas import tpu_sc as plsc

N = 10000
E = 160000
D_IN = 256
H = 512
F = 128
NC = 2
NS = 16
NP = 10240
ROWS_PER_SUB = NP // NS
EP = 163840
CH = 128
NCH_D = EP // (NC * NS) // CH
NB = 10
BSH = 10
BR = 1 << BSH
AR = 1152
PCAP = 112
EPW = EP // (NC * NS)
VEC = 16

_mesh = plsc.VectorSubcoreMesh(
    core_axis_name="c", subcore_axis_name="s", num_cores=NC, num_subcores=NS
)

_cp = pltpu.CompilerParams()
if "needs_layout_passes" in pltpu.CompilerParams.__dataclass_fields__:
    _cp = dataclasses.replace(_cp, needs_layout_passes=False)


@functools.partial(
    pl.kernel,
    out_type=jax.ShapeDtypeStruct((NC, NP, F), jnp.float32),
    mesh=_mesh,
    scratch_types=[
        pltpu.VMEM((NCH_D, CH), jnp.int32),
        pltpu.VMEM((CH, F), jnp.float32),
        pltpu.VMEM_SHARED((NP, F), jnp.float32),
        pltpu.SemaphoreType.DMA,
    ],
)
def _sc_degree(dst_hbm, ones_hbm, zeros_hbm, out_hbm, dstm, onesv, acc, sem):
    c = lax.axis_index("c")
    s = lax.axis_index("s")
    pltpu.sync_copy(zeros_hbm, acc.at[pl.ds(s * ROWS_PER_SUB, ROWS_PER_SUB)])
    pltpu.sync_copy(ones_hbm, onesv)
    pltpu.sync_copy(dst_hbm.at[c * NS + s], dstm)
    plsc.subcore_barrier()
    @pl.loop(0, NCH_D)
    def _(i):
        pltpu.async_copy(onesv, acc.at[dstm.at[i]], sem, add=True)
    @pl.loop(0, NCH_D)
    def _(i):
        pltpu.make_async_copy(onesv, acc.at[dstm.at[i]], sem).wait()
    plsc.subcore_barrier()
    sl = pl.ds(s * ROWS_PER_SUB, ROWS_PER_SUB)
    pltpu.sync_copy(acc.at[sl], out_hbm.at[c].at[sl])


@functools.partial(
    pl.kernel,
    out_type=[
        jax.ShapeDtypeStruct((NC * NS, PCAP * CH), jnp.int32),
        jax.ShapeDtypeStruct((NC * NS, PCAP, CH), jnp.int32),
        jax.ShapeDtypeStruct((NC * NS, 2, VEC), jnp.int32),
    ],
    mesh=_mesh,
    scratch_types=[
        pltpu.VMEM((NCH_D, CH), jnp.int32),
        pltpu.VMEM((NCH_D, CH), jnp.int32),
        pltpu.VMEM((PCAP * CH,), jnp.int32),
        pltpu.VMEM((PCAP * CH,), jnp.int32),
        pltpu.VMEM((PCAP, CH), jnp.int32),
        pltpu.VMEM((2, VEC), jnp.int32),
        pltpu.VMEM((VEC,), jnp.int32),
        pltpu.VMEM((VEC,), jnp.int32),
    ],
    compiler_params=_cp,
)
def _sc_bin(src_hbm, dst_hbm, bsrc_hbm, brloc_hbm, meta_hbm,
            srcm, dstm, locs, locr, locr2, metav, cntv, curv):
    c = lax.axis_index("c")
    s = lax.axis_index("s")
    w = c * NS + s
    pltpu.sync_copy(src_hbm.at[w], srcm)
    pltpu.sync_copy(dst_hbm.at[w], dstm)
    lanes = lax.iota(jnp.int32, VEC)

    @pl.loop(0, PCAP * (CH // VEC))
    def _(i):
        locs.at[pl.ds(i * VEC, VEC)][...] = jnp.zeros((VEC,), jnp.int32)
        locr.at[pl.ds(i * VEC, VEC)][...] = BR + lanes

    cntv[...] = jnp.zeros((VEC,), jnp.int32)
    @pl.loop(0, EPW // VEC)
    def _(i):
        r = i // (CH // VEC)
        col = (i % (CH // VEC)) * VEC
        v = dstm.at[r, pl.ds(col, VEC)][...]
        b_of = lax.shift_right_logical(v, BSH)
        cnts = cntv[...]
        for b in range(NB):
            pop = plsc.all_reduce_population_count(b_of == b)
            cnts = jnp.where(lanes == b, cnts + pop, cnts)
        cntv[...] = cnts

    cnts = cntv[...]
    nch = lax.shift_right_logical(cnts + (CH - 1), 7)
    rnd8 = jnp.where(lanes < NB,
                     lax.shift_left(lax.shift_right_logical(nch + 7, 3), 3), 0)
    off8 = plsc.cumsum(rnd8) - rnd8
    metav.at[0, pl.ds(0, VEC)][...] = nch
    metav.at[1, pl.ds(0, VEC)][...] = off8
    curv[...] = jnp.where(lanes < NB, off8 * CH, 0)

    @pl.loop(0, EPW // VEC)
    def _(i):
        r = i // (CH // VEC)
        col = (i % (CH // VEC)) * VEC
        v = dstm.at[r, pl.ds(col, VEC)][...]
        sv = srcm.at[r, pl.ds(col, VEC)][...]
        b_of = lax.shift_right_logical(v, BSH)
        rloc = v - lax.shift_left(b_of, BSH)
        cur = curv[...]
        for b in range(NB):
            m = b_of == b
            base = jnp.max(jnp.where(lanes == b, cur, 0))
            cs = plsc.cumsum(m.astype(jnp.int32))
            idx = base + cs - 1
            plsc.store_scatter(locs, (idx,), sv, mask=m)
            plsc.store_scatter(locr, (idx,), rloc, mask=m)
            pop = plsc.all_reduce_population_count(m)
            cur = jnp.where(lanes == b, cur + pop, cur)
        curv[...] = cur

    @pl.loop(0, PCAP * (CH // VEC))
    def _(i):
        r = i // (CH // VEC)
        col = (i % (CH // VEC)) * VEC
        locr2.at[r, pl.ds(col, VEC)][...] = locr.at[pl.ds(i * VEC, VEC)][...]

    pltpu.sync_copy(locs, bsrc_hbm.at[w])
    pltpu.sync_copy(locr2, brloc_hbm.at[w])
    pltpu.sync_copy(metav, meta_hbm.at[w])


WIN = 8


@functools.partial(
    pl.kernel,
    out_type=jax.ShapeDtypeStruct((NC, NP, F), jnp.float32),
    mesh=_mesh,
    scratch_types=[
        pltpu.VMEM((WIN * CH,), jnp.int32),
        pltpu.VMEM((WIN, CH), jnp.int32),
        pltpu.VMEM((2, 2, VEC), jnp.int32),
        pltpu.VMEM((CH, F), jnp.float32),
        pltpu.VMEM((CH, F), jnp.float32),
        pltpu.VMEM_SHARED((NP, F), jnp.float32),
        pltpu.VMEM_SHARED((AR, F), jnp.float32),
        pltpu.SemaphoreType.DMA,
        pltpu.SemaphoreType.DMA,
    ],
    compiler_params=_cp,
)
def _sc_edge_pass(y_hbm, bsrc_hbm, brloc_hbm, meta_hbm, zeros_hbm, out_hbm,
                  srcw, rlocw, metv, rows0, rows1, ysp, acc, sem0, sem1):
    c = lax.axis_index("c")
    s = lax.axis_index("s")
    lanes = lax.iota(jnp.int32, VEC)

    sl = pl.ds(s * ROWS_PER_SUB, ROWS_PER_SUB)
    pltpu.sync_copy(y_hbm.at[c].at[sl], ysp.at[sl])
    pltpu.sync_copy(meta_hbm.at[pl.ds(2 * s, 2)], metv)

    @pl.loop(0, NB)
    def _(b):
        pltpu.sync_copy(zeros_hbm, acc.at[pl.ds(s * (AR // NS), AR // NS)])
        plsc.subcore_barrier()
        for pi in range(2):
            p = 2 * s + pi
            nchv = metv.at[pi, 0, pl.ds(0, VEC)][...]
            offv = metv.at[pi, 1, pl.ds(0, VEC)][...]
            nch = jnp.max(jnp.where(lanes == b, nchv, 0))
            offc = pl.multiple_of(jnp.max(jnp.where(lanes == b, offv, 0)), 8)
            @pl.loop(0, 5)
            def _(win):
                @pl.when(nch > win * WIN)
                def _():
                    pltpu.sync_copy(
                        bsrc_hbm.at[p].at[pl.ds((offc + win * WIN) * CH,
                                                WIN * CH)], srcw)
                    pltpu.sync_copy(
                        brloc_hbm.at[p].at[pl.ds(offc + win * WIN, WIN)],
                        rlocw)
                    cnt = jnp.minimum(nch - win * WIN, WIN)

                    def g_start(k, buf, sem):
                        pltpu.async_copy(
                            ysp.at[srcw.at[pl.ds(k * CH, CH)]], buf, sem)

                    def g_wait(k, buf, sem):
                        pltpu.make_async_copy(
                            ysp.at[srcw.at[pl.ds(k * CH, CH)]], buf,
                            sem).wait()

                    g_start(0, rows0, sem0)
                    @pl.loop(0, WIN // 2)
                    def _(j):
                        k = 2 * j
                        @pl.when(k < cnt)
                        def _():
                            @pl.when(k + 1 < cnt)
                            def _():
                                g_start(k + 1, rows1, sem1)
                            g_wait(k, rows0, sem0)
                            pltpu.sync_copy(rows0, acc.at[rlocw.at[k]],
                                            add=True)
                            @pl.when(k + 2 < cnt)
                            def _():
                                g_start(k + 2, rows0, sem0)
                            @pl.when(k + 1 < cnt)
                            def _():
                                g_wait(k + 1, rows1, sem1)
                                pltpu.sync_copy(rows1, acc.at[rlocw.at[k + 1]],
                                                add=True)
        plsc.subcore_barrier()
        rps = BR // NS
        pltpu.sync_copy(acc.at[pl.ds(s * rps, rps)],
                        out_hbm.at[c].at[pl.ds(pl.multiple_of(b * BR + s * rps, 8), rps)])
        plsc.subcore_barrier()


BN = BR


def _dinv_block(cnt_ref):
    deg = cnt_ref[0, :, 0] + cnt_ref[1, :, 0] + 2.0
    return lax.rsqrt(deg)[:, None]


def _tc_scale_body(cnt_ref, x_ref, y_ref):
    dinv = _dinv_block(cnt_ref)
    y_ref[0] = dinv * x_ref[:, :F]
    y_ref[1] = dinv * x_ref[:, F:]


def _tc_mid_body(cnt_ref, s1_ref, x_ref, w1_ref, b1_ref, w2_ref,
                 y2_ref, h2_ref):
    dinv = _dinv_block(cnt_ref)
    sfull = jnp.concatenate([s1_ref[0], s1_ref[1]], axis=1)
    xa = dinv * sfull + (2.0 * dinv * dinv) * x_ref[...]
    t = jnp.dot(xa, w1_ref[...], preferred_element_type=jnp.float32)
    t = t + b1_ref[...]
    t = jnp.where(t > 0.0, t, jnp.exp(jnp.minimum(t, 0.0)) - 1.0)
    h2 = jnp.dot(t, w2_ref[...], preferred_element_type=jnp.float32)
    h2_ref[...] = h2
    y2 = dinv * h2
    y2_ref[0] = y2[:, :F]
    y2_ref[1] = y2[:, F:]


def _tc_final_body(cnt_ref, s2_ref, h2_ref, b2_ref, out_ref):
    dinv = _dinv_block(cnt_ref)
    sfull = jnp.concatenate([s2_ref[0], s2_ref[1]], axis=1)
    out_ref[...] = dinv * sfull + (2.0 * dinv * dinv) * h2_ref[...] + b2_ref[...]


def _cnt_spec():
    return pl.BlockSpec((NC, BN, F), lambda i: (0, i, 0))


def _s_spec():
    return pl.BlockSpec((NC, BN, F), lambda i: (0, i, 0))


@jax.jit
def kernel(x, edge_index, W1, b1, W2, b2):
    src = edge_index[0].astype(jnp.int32)
    dst = edge_index[1].astype(jnp.int32)
    pad = EP - E
    srcp = jnp.concatenate([src, jnp.zeros((pad,), jnp.int32)])
    junk = N + jnp.arange(pad, dtype=jnp.int32) % (NP - N)
    dstp = jnp.concatenate([dst, junk])
    src_d = srcp.reshape(NC * NS, NCH_D, CH)
    dst_d = dstp.reshape(NC * NS, NCH_D, CH)

    xp = jnp.zeros((NP, D_IN), x.dtype).at[:N].set(x)

    onesF = jnp.ones((CH, F), jnp.float32)
    zerosF = jnp.zeros((ROWS_PER_SUB, F), jnp.float32)
    zerosA = jnp.zeros((AR // NS, F), jnp.float32)

    cnt = _sc_degree(dst_d, onesF, zerosF)
    bsrc, brloc, meta = _sc_bin(src_d, dst_d)

    y = pl.pallas_call(
        _tc_scale_body,
        grid=(NB,),
        in_specs=[_cnt_spec(), pl.BlockSpec((BN, D_IN), lambda i: (i, 0))],
        out_specs=pl.BlockSpec((NC, BN, F), lambda i: (0, i, 0)),
        out_shape=jax.ShapeDtypeStruct((NC, NP, F), jnp.float32),
    )(cnt, xp)

    s1 = _sc_edge_pass(y, bsrc, brloc, meta, zerosA)

    y2, h2 = pl.pallas_call(
        _tc_mid_body,
        grid=(NB,),
        in_specs=[
            _cnt_spec(),
            _s_spec(),
            pl.BlockSpec((BN, D_IN), lambda i: (i, 0)),
            pl.BlockSpec((D_IN, H), lambda i: (0, 0)),
            pl.BlockSpec((1, H), lambda i: (0, 0)),
            pl.BlockSpec((H, D_IN), lambda i: (0, 0)),
        ],
        out_specs=[
            pl.BlockSpec((NC, BN, F), lambda i: (0, i, 0)),
            pl.BlockSpec((BN, D_IN), lambda i: (i, 0)),
        ],
        out_shape=[
            jax.ShapeDtypeStruct((NC, NP, F), jnp.float32),
            jax.ShapeDtypeStruct((NP, D_IN), jnp.float32),
        ],
    )(cnt, s1, xp, W1, b1.reshape(1, H), W2)

    s2 = _sc_edge_pass(y2, bsrc, brloc, meta, zerosA)

    out = pl.pallas_call(
        _tc_final_body,
        grid=(NB,),
        in_specs=[
            _cnt_spec(),
            _s_spec(),
            pl.BlockSpec((BN, D_IN), lambda i: (i, 0)),
            pl.BlockSpec((1, D_IN), lambda i: (0, 0)),
        ],
        out_specs=pl.BlockSpec((BN, D_IN), lambda i: (i, 0)),
        out_shape=jax.ShapeDtypeStruct((NP, D_IN), jnp.float32),
    )(cnt, s2, h2, b2.reshape(1, D_IN))
    return out[:N]
